# baseline (speedup 1.0000x reference)
import functools

import jax
import jax.numpy as jnp
from jax import lax
from jax.experimental import pallas as pl
from jax.experimental.pallas import tpu as pltpu
from jax.experimental.pallas import tpu_sc as plsc

V = 100000
D = 64
H = 128
C0 = 6667
C1 = 20001
S = 50
B = 32
HEAD = C0 + 2
N = S * B

_SC_ROWS = 64
_SC_WORKERS = N // _SC_ROWS


def _gather_emb(emb2, idx_e2):
    mesh = plsc.VectorSubcoreMesh(core_axis_name="c", subcore_axis_name="s")

    @functools.partial(
        pl.kernel,
        mesh=mesh,
        out_type=jax.ShapeDtypeStruct((N, 2 * D), jnp.float32),
        scratch_types=[
            pltpu.VMEM((_SC_ROWS,), jnp.int32),
            pltpu.VMEM((_SC_ROWS, 2 * D), jnp.float32),
            pltpu.SemaphoreType.DMA,
        ],
    )
    def gather_kernel(emb_h, ie_h, x_o, ie_v, x_v, sem):
        wid = lax.axis_index("s") * 2 + lax.axis_index("c")

        @pl.when(wid < _SC_WORKERS)
        def _():
            base = wid * _SC_ROWS
            pltpu.sync_copy(ie_h.at[pl.ds(base, _SC_ROWS)], ie_v)
            pltpu.async_copy(emb_h.at[ie_v], x_v, sem).wait()
            pltpu.sync_copy(x_v, x_o.at[pl.ds(base, _SC_ROWS)])

    return gather_kernel(emb2, idx_e2)


def _gather_targets(head_W, idx_h, t0_pack, idx_0p):
    mesh = plsc.VectorSubcoreMesh(core_axis_name="c", subcore_axis_name="s")

    @functools.partial(
        pl.kernel,
        mesh=mesh,
        out_type=[
            jax.ShapeDtypeStruct((N, H), jnp.float32),
            jax.ShapeDtypeStruct((N, H), jnp.float32),
        ],
        scratch_types=[
            pltpu.VMEM((_SC_ROWS,), jnp.int32),
            pltpu.VMEM((_SC_ROWS,), jnp.int32),
            pltpu.VMEM((_SC_ROWS, H), jnp.float32),
            pltpu.VMEM((_SC_ROWS, H), jnp.float32),
            pltpu.SemaphoreType.DMA,
        ],
    )
    def gather_kernel(hw_h, ih_h, t0_h, i0_h, wh_o, w0_o,
                      ih_v, i0_v, wh_v, w0_v, sem):
        wid = lax.axis_index("s") * 2 + lax.axis_index("c")

        @pl.when(wid < _SC_WORKERS)
        def _():
            base = wid * _SC_ROWS
            c1 = pltpu.async_copy(ih_h.at[pl.ds(base, _SC_ROWS)], ih_v, sem)
            c2 = pltpu.async_copy(i0_h.at[pl.ds(base, _SC_ROWS)], i0_v, sem)
            c1.wait()
            c2.wait()
            g1 = pltpu.async_copy(hw_h.at[ih_v], wh_v, sem)
            g2 = pltpu.async_copy(t0_h.at[i0_v], w0_v, sem)
            g1.wait()
            g2.wait()
            pltpu.sync_copy(wh_v, wh_o.at[pl.ds(base, _SC_ROWS)])
            pltpu.sync_copy(w0_v, w0_o.at[pl.ds(base, _SC_ROWS)])

    return gather_kernel(head_W, idx_h, t0_pack, idx_0p)


def _gather_tail1(t1_pack, idx_1q):
    mesh = plsc.VectorSubcoreMesh(core_axis_name="c", subcore_axis_name="s")

    @functools.partial(
        pl.kernel,
        mesh=mesh,
        out_type=jax.ShapeDtypeStruct((N, H), jnp.float32),
        scratch_types=[
            pltpu.VMEM((_SC_ROWS,), jnp.int32),
            pltpu.VMEM((_SC_ROWS, H), jnp.float32),
            pltpu.SemaphoreType.DMA,
        ],
    )
    def gather_kernel(t1_h, i1_h, w1_o, i1_v, w1_v, sem):
        wid = lax.axis_index("s") * 2 + lax.axis_index("c")

        @pl.when(wid < _SC_WORKERS)
        def _():
            base = wid * _SC_ROWS
            pltpu.sync_copy(i1_h.at[pl.ds(base, _SC_ROWS)], i1_v)
            pltpu.async_copy(t1_h.at[i1_v], w1_v, sem).wait()
            pltpu.sync_copy(w1_v, w1_o.at[pl.ds(base, _SC_ROWS)])

    return gather_kernel(t1_pack, idx_1q)


def _lstm_body(xg_ref, par_ref, wih_ref, whh_ref, b_ref, p0_ref, p1_ref,
               h2_ref, flat_ref, pr0_ref, pr1_ref, hd2_ref, xp_ref):
    xg = xg_ref[...]
    x = jnp.where(par_ref[...] == 1, xg[:, D:], xg[:, :D])
    xp_ref[...] = lax.dot_general(
        x, wih_ref[...], (((1,), (1,)), ((), ())),
        preferred_element_type=jnp.float32) + b_ref[...]

    def step(t, carry):
        h, c = carry
        z = xp_ref[pl.ds(t * B, B), :] + lax.dot_general(
            h, whh_ref[...], (((1,), (1,)), ((), ())),
            preferred_element_type=jnp.float32)
        i = jax.nn.sigmoid(z[:, :H])
        f = jax.nn.sigmoid(z[:, H:2 * H])
        g = jnp.tanh(z[:, 2 * H:3 * H])
        o = jax.nn.sigmoid(z[:, 3 * H:])
        c = f * c + i * g
        h = o * jnp.tanh(c)
        flat_ref[pl.ds(t * B, B), :] = h
        return (h, c)

    lax.fori_loop(0, S, step,
                  (jnp.zeros((B, H), jnp.float32),
                   jnp.zeros((B, H), jnp.float32)))
    flat = flat_ref[...]
    pr0_ref[...] = lax.dot_general(flat, p0_ref[...], (((1,), (1,)), ((), ())),
                                   preferred_element_type=jnp.float32)
    pr1_ref[...] = lax.dot_general(flat, p1_ref[...], (((1,), (1,)), ((), ())),
                                   preferred_element_type=jnp.float32)
    hd2_ref[...] = lax.dot_general(flat, h2_ref[...], (((1,), (1,)), ((), ())),
                                   preferred_element_type=jnp.float32)


def _lstm(xg, par, W_ih, W_hh, b2, tail0_proj, tail1_proj, headW2):
    return pl.pallas_call(
        _lstm_body,
        out_shape=[
            jax.ShapeDtypeStruct((N, H), jnp.float32),
            jax.ShapeDtypeStruct((N, H // 2), jnp.float32),
            jax.ShapeDtypeStruct((N, H // 4), jnp.float32),
            jax.ShapeDtypeStruct((N, 2), jnp.float32),
        ],
        scratch_shapes=[pltpu.VMEM((N, 4 * H), jnp.float32)],
    )(xg, par, W_ih, W_hh, b2, tail0_proj, tail1_proj, headW2)


def _lse_sum(proj, w, rows, chunk):
    grid = -(-rows // chunk)
    last = rows - (grid - 1) * chunk
    k_dim = proj.shape[1]

    def body(p_ref, w_ref, s_ref):
        i = pl.program_id(0)

        @pl.when(i == 0)
        def _():
            s_ref[...] = jnp.zeros_like(s_ref)

        logits = lax.dot_general(
            p_ref[...].astype(jnp.bfloat16), w_ref[...].astype(jnp.bfloat16),
            (((1,), (1,)), ((), ())),
            preferred_element_type=jnp.float32)
        e = jnp.exp(logits)
        if last == chunk:
            s_ref[...] += jnp.sum(e, axis=1, keepdims=True)
        else:
            @pl.when(i < grid - 1)
            def _():
                s_ref[...] += jnp.sum(e, axis=1, keepdims=True)

            @pl.when(i == grid - 1)
            def _():
                col = lax.broadcasted_iota(jnp.int32, e.shape, 1)
                s_ref[...] += jnp.sum(jnp.where(col < last, e, 0.0),
                                      axis=1, keepdims=True)

    return pl.pallas_call(
        body,
        grid=(grid,),
        in_specs=[
            pl.BlockSpec((N, k_dim), lambda i: (0, 0)),
            pl.BlockSpec((chunk, k_dim), lambda i: (i, 0)),
        ],
        out_specs=pl.BlockSpec((N, 1), lambda i: (0, 0)),
        out_shape=jax.ShapeDtypeStruct((N, 1), jnp.float32),
        compiler_params=pltpu.CompilerParams(
            dimension_semantics=("arbitrary",)),
    )(proj, w)


def _assemble_body(tgt_ref, sh_ref, s0_ref, s1_ref, hd2_ref, fl_ref, wh_ref,
                   p0_ref, w0_ref, p1_ref, w1_ref, o_ref):
    tgt = tgt_ref[...]
    lse_h = jnp.log(sh_ref[...])
    lse0 = jnp.log(s0_ref[...])
    lse1 = jnp.log(s1_ref[...] - 1.0)
    w0g = w0_ref[...]
    w0 = jnp.where((tgt - C0) % 2 == 1, w0g[:, H // 2:], w0g[:, :H // 2])
    w1g = w1_ref[...]
    q = (tgt - C1) % 4
    Q = H // 4
    w1 = jnp.where(q == 0, w1g[:, :Q],
                   jnp.where(q == 1, w1g[:, Q:2 * Q],
                             jnp.where(q == 2, w1g[:, 2 * Q:3 * Q],
                                       w1g[:, 3 * Q:])))
    th = jnp.sum(fl_ref[...] * wh_ref[...], axis=1, keepdims=True)
    t0 = jnp.sum(p0_ref[...] * w0, axis=1, keepdims=True)
    t1 = jnp.sum(p1_ref[...] * w1, axis=1, keepdims=True)
    hd2 = hd2_ref[...]
    out = jnp.where(tgt < C0, th - lse_h, 0.0)
    out = jnp.where((tgt >= C0) & (tgt < C1),
                    hd2[:, 0:1] - lse_h + t0 - lse0, out)
    out = jnp.where(tgt >= C1, hd2[:, 1:2] - lse_h + t1 - lse1, out)
    o_ref[...] = jnp.full((1, 1), -1.0 / N, jnp.float32) * jnp.sum(out)


def _assemble(tgt2, s_h, s_0, s_1, hd2, flat, wh, pr0, w0, pr1, w1):
    return pl.pallas_call(
        _assemble_body,
        out_shape=jax.ShapeDtypeStruct((1, 1), jnp.float32),
    )(tgt2, s_h, s_0, s_1, hd2, flat, wh, pr0, w0, pr1, w1)


def kernel(review_input, review_output, emb, W_ih, W_hh, b_ih, b_hh,
           head_W, tail0_proj, tail0_out, tail1_proj, tail1_out):
    ie = review_input.reshape(-1).astype(jnp.int32)
    tgt = review_output.reshape(-1).astype(jnp.int32)
    ih = jnp.clip(tgt, 0, C0 - 1)
    i0 = jnp.clip(tgt - C0, 0, C1 - C0 - 1)
    i1 = jnp.clip(tgt - C1, 0, V - C1 - 1)

    emb2 = emb.reshape(V // 2, 2 * D)
    xg = _gather_emb(emb2, ie // 2)
    t0_pack = tail0_out.reshape((C1 - C0) // 2, H)
    wh, w0 = _gather_targets(head_W, ih, t0_pack, i0 // 2)
    t1_pad = jnp.concatenate(
        [tail1_out, jnp.zeros((1, H // 4), jnp.float32)], axis=0)
    w1 = _gather_tail1(t1_pad.reshape((V - C1 + 1) // 4, H), i1 // 4)

    b2 = (b_ih + b_hh).reshape(1, 4 * H)
    headW2 = lax.slice(head_W, (C0, 0), (C0 + 2, H))
    par = (ie % 2).reshape(N, 1)
    flat, pr0, pr1, hd2 = _lstm(xg, par, W_ih, W_hh, b2, tail0_proj,
                                tail1_proj, headW2)

    s_h = _lse_sum(flat, head_W, HEAD, 2048)
    s_0 = _lse_sum(pr0, tail0_out, C1 - C0, 2048)
    s_1 = _lse_sum(pr1, t1_pad, V - C1 + 1, 2000)

    loss = _assemble(tgt.reshape(N, 1), s_h, s_0, s_1, hd2, flat, wh,
                     pr0, w0, pr1, w1)
    return loss.reshape(())


# speedup vs baseline: 1.1925x; 1.1925x over previous
import functools

import jax
import jax.numpy as jnp
from jax import lax
from jax.experimental import pallas as pl
from jax.experimental.pallas import tpu as pltpu
from jax.experimental.pallas import tpu_sc as plsc

V = 100000
D = 64
H = 128
C0 = 6667
C1 = 20001
S = 50
B = 32
HEAD = C0 + 2
N = S * B

_SC_ROWS = 64
_SC_WORKERS = N // _SC_ROWS


def _gather_emb(emb2, idx_e2):
    mesh = plsc.VectorSubcoreMesh(core_axis_name="c", subcore_axis_name="s")

    @functools.partial(
        pl.kernel,
        mesh=mesh,
        out_type=jax.ShapeDtypeStruct((N, 2 * D), jnp.float32),
        scratch_types=[
            pltpu.VMEM((_SC_ROWS,), jnp.int32),
            pltpu.VMEM((_SC_ROWS, 2 * D), jnp.float32),
            pltpu.SemaphoreType.DMA,
        ],
    )
    def gather_kernel(emb_h, ie_h, x_o, ie_v, x_v, sem):
        wid = lax.axis_index("s") * 2 + lax.axis_index("c")

        @pl.when(wid < _SC_WORKERS)
        def _():
            base = wid * _SC_ROWS
            pltpu.sync_copy(ie_h.at[pl.ds(base, _SC_ROWS)], ie_v)
            pltpu.async_copy(emb_h.at[ie_v], x_v, sem).wait()
            pltpu.sync_copy(x_v, x_o.at[pl.ds(base, _SC_ROWS)])

    return gather_kernel(emb2, idx_e2)


def _gather_targets(head_W, idx_h, t0_pack, idx_0p):
    mesh = plsc.VectorSubcoreMesh(core_axis_name="c", subcore_axis_name="s")

    @functools.partial(
        pl.kernel,
        mesh=mesh,
        out_type=[
            jax.ShapeDtypeStruct((N, H), jnp.float32),
            jax.ShapeDtypeStruct((N, H), jnp.float32),
        ],
        scratch_types=[
            pltpu.VMEM((_SC_ROWS,), jnp.int32),
            pltpu.VMEM((_SC_ROWS,), jnp.int32),
            pltpu.VMEM((_SC_ROWS, H), jnp.float32),
            pltpu.VMEM((_SC_ROWS, H), jnp.float32),
            pltpu.SemaphoreType.DMA,
        ],
    )
    def gather_kernel(hw_h, ih_h, t0_h, i0_h, wh_o, w0_o,
                      ih_v, i0_v, wh_v, w0_v, sem):
        wid = lax.axis_index("s") * 2 + lax.axis_index("c")

        @pl.when(wid < _SC_WORKERS)
        def _():
            base = wid * _SC_ROWS
            c1 = pltpu.async_copy(ih_h.at[pl.ds(base, _SC_ROWS)], ih_v, sem)
            c2 = pltpu.async_copy(i0_h.at[pl.ds(base, _SC_ROWS)], i0_v, sem)
            c1.wait()
            c2.wait()
            g1 = pltpu.async_copy(hw_h.at[ih_v], wh_v, sem)
            g2 = pltpu.async_copy(t0_h.at[i0_v], w0_v, sem)
            g1.wait()
            g2.wait()
            pltpu.sync_copy(wh_v, wh_o.at[pl.ds(base, _SC_ROWS)])
            pltpu.sync_copy(w0_v, w0_o.at[pl.ds(base, _SC_ROWS)])

    return gather_kernel(head_W, idx_h, t0_pack, idx_0p)


def _gather_tail1(t1_out, idx_1):
    mesh = plsc.VectorSubcoreMesh(core_axis_name="c", subcore_axis_name="s")

    @functools.partial(
        pl.kernel,
        mesh=mesh,
        out_type=jax.ShapeDtypeStruct((N, H // 4), jnp.float32),
        scratch_types=[
            pltpu.VMEM((_SC_ROWS,), jnp.int32),
            pltpu.VMEM((_SC_ROWS, H // 4), jnp.float32),
            pltpu.SemaphoreType.DMA,
        ],
        compiler_params=pltpu.CompilerParams(use_tc_tiling_on_sc=False),
    )
    def gather_kernel(t1_h, i1_h, w1_o, i1_v, w1_v, sem):
        wid = lax.axis_index("s") * 2 + lax.axis_index("c")

        @pl.when(wid < _SC_WORKERS)
        def _():
            base = wid * _SC_ROWS
            pltpu.sync_copy(i1_h.at[pl.ds(base, _SC_ROWS)], i1_v)
            pltpu.async_copy(t1_h.at[i1_v], w1_v, sem).wait()
            pltpu.sync_copy(w1_v, w1_o.at[pl.ds(base, _SC_ROWS)])

    return gather_kernel(t1_out, idx_1)


def _lstm_body(xg_ref, par_ref, wih_ref, whh_ref, b_ref, p0_ref, p1_ref,
               h2_ref, flat_ref, pr0_ref, pr1_ref, hd2_ref, xp_ref):
    xg = xg_ref[...]
    x = jnp.where(par_ref[...] == 1, xg[:, D:], xg[:, :D])
    xp = lax.dot_general(
        x.astype(jnp.bfloat16), wih_ref[...].astype(jnp.bfloat16),
        (((1,), (1,)), ((), ())),
        preferred_element_type=jnp.float32) + b_ref[...]
    xp_ref[...] = xp.reshape(S, B, 4 * H)
    whh_b = whh_ref[...].astype(jnp.bfloat16)

    def step(t, carry):
        h, c = carry
        z = xp_ref[t] + lax.dot_general(
            h.astype(jnp.bfloat16), whh_b, (((1,), (1,)), ((), ())),
            preferred_element_type=jnp.float32)
        i = jax.nn.sigmoid(z[:, :H])
        f = jax.nn.sigmoid(z[:, H:2 * H])
        g = jnp.tanh(z[:, 2 * H:3 * H])
        o = jax.nn.sigmoid(z[:, 3 * H:])
        c = f * c + i * g
        h = o * jnp.tanh(c)
        flat_ref[t] = h
        return (h, c)

    lax.fori_loop(0, S, step,
                  (jnp.zeros((B, H), jnp.float32),
                   jnp.zeros((B, H), jnp.float32)),
                  unroll=2)
    flat = flat_ref[...].reshape(N, H)
    pr0_ref[...] = lax.dot_general(flat, p0_ref[...], (((1,), (1,)), ((), ())),
                                   preferred_element_type=jnp.float32)
    pr1_ref[...] = lax.dot_general(flat, p1_ref[...], (((1,), (1,)), ((), ())),
                                   preferred_element_type=jnp.float32)
    hd2_ref[...] = lax.dot_general(flat, h2_ref[...], (((1,), (1,)), ((), ())),
                                   preferred_element_type=jnp.float32)


def _lstm(xg, par, W_ih, W_hh, b2, tail0_proj, tail1_proj, headW2):
    return pl.pallas_call(
        _lstm_body,
        out_shape=[
            jax.ShapeDtypeStruct((S, B, H), jnp.float32),
            jax.ShapeDtypeStruct((N, H // 2), jnp.float32),
            jax.ShapeDtypeStruct((N, H // 4), jnp.float32),
            jax.ShapeDtypeStruct((N, 2), jnp.float32),
        ],
        scratch_shapes=[pltpu.VMEM((S, B, 4 * H), jnp.float32)],
    )(xg, par, W_ih, W_hh, b2, tail0_proj, tail1_proj, headW2)


def _lse_sum(proj, w, rows, chunk):
    grid = -(-rows // chunk)
    last = rows - (grid - 1) * chunk
    k_dim = proj.shape[1]

    def body(p_ref, w_ref, s_ref):
        i = pl.program_id(0)

        @pl.when(i == 0)
        def _():
            s_ref[...] = jnp.zeros_like(s_ref)

        logits = lax.dot_general(
            p_ref[...].astype(jnp.bfloat16), w_ref[...].astype(jnp.bfloat16),
            (((1,), (1,)), ((), ())),
            preferred_element_type=jnp.float32)
        e = jnp.exp(logits)
        if last == chunk:
            s_ref[...] += jnp.sum(e, axis=1, keepdims=True)
        else:
            @pl.when(i < grid - 1)
            def _():
                s_ref[...] += jnp.sum(e, axis=1, keepdims=True)

            @pl.when(i == grid - 1)
            def _():
                col = lax.broadcasted_iota(jnp.int32, e.shape, 1)
                s_ref[...] += jnp.sum(jnp.where(col < last, e, 0.0),
                                      axis=1, keepdims=True)

    return pl.pallas_call(
        body,
        grid=(grid,),
        in_specs=[
            pl.BlockSpec((N, k_dim), lambda i: (0, 0)),
            pl.BlockSpec((chunk, k_dim), lambda i: (i, 0)),
        ],
        out_specs=pl.BlockSpec((N, 1), lambda i: (0, 0)),
        out_shape=jax.ShapeDtypeStruct((N, 1), jnp.float32),
        compiler_params=pltpu.CompilerParams(
            dimension_semantics=("arbitrary",)),
    )(proj, w)


def _assemble_body(tgt_ref, sh_ref, s0_ref, s1_ref, hd2_ref, fl_ref, wh_ref,
                   p0_ref, w0_ref, p1_ref, w1_ref, o_ref):
    tgt = tgt_ref[...]
    lse_h = jnp.log(sh_ref[...])
    lse0 = jnp.log(s0_ref[...])
    lse1 = jnp.log(s1_ref[...])
    w0g = w0_ref[...]
    w0 = jnp.where((tgt - C0) % 2 == 1, w0g[:, H // 2:], w0g[:, :H // 2])
    th = jnp.sum(fl_ref[...] * wh_ref[...], axis=1, keepdims=True)
    t0 = jnp.sum(p0_ref[...] * w0, axis=1, keepdims=True)
    t1 = jnp.sum(p1_ref[...] * w1_ref[...], axis=1, keepdims=True)
    hd2 = hd2_ref[...]
    out = jnp.where(tgt < C0, th - lse_h, 0.0)
    out = jnp.where((tgt >= C0) & (tgt < C1),
                    hd2[:, 0:1] - lse_h + t0 - lse0, out)
    out = jnp.where(tgt >= C1, hd2[:, 1:2] - lse_h + t1 - lse1, out)
    o_ref[...] = jnp.full((1, 1), -1.0 / N, jnp.float32) * jnp.sum(out)


def _assemble(tgt2, s_h, s_0, s_1, hd2, flat, wh, pr0, w0, pr1, w1):
    return pl.pallas_call(
        _assemble_body,
        out_shape=jax.ShapeDtypeStruct((1, 1), jnp.float32),
    )(tgt2, s_h, s_0, s_1, hd2, flat, wh, pr0, w0, pr1, w1)


def kernel(review_input, review_output, emb, W_ih, W_hh, b_ih, b_hh,
           head_W, tail0_proj, tail0_out, tail1_proj, tail1_out):
    ie = review_input.reshape(-1).astype(jnp.int32)
    tgt = review_output.reshape(-1).astype(jnp.int32)
    ih = jnp.clip(tgt, 0, C0 - 1)
    i0 = jnp.clip(tgt - C0, 0, C1 - C0 - 1)
    i1 = jnp.clip(tgt - C1, 0, V - C1 - 1)

    emb2 = emb.reshape(V // 2, 2 * D)
    xg = _gather_emb(emb2, ie // 2)
    t0_pack = tail0_out.reshape((C1 - C0) // 2, H)
    wh, w0 = _gather_targets(head_W, ih, t0_pack, i0 // 2)
    w1 = _gather_tail1(tail1_out, i1)

    b2 = (b_ih + b_hh).reshape(1, 4 * H)
    headW2 = lax.slice(head_W, (C0, 0), (C0 + 2, H))
    par = (ie % 2).reshape(N, 1)
    flat3, pr0, pr1, hd2 = _lstm(xg, par, W_ih, W_hh, b2, tail0_proj,
                                 tail1_proj, headW2)
    flat = flat3.reshape(N, H)

    s_h = _lse_sum(flat, head_W, HEAD, 2048)
    s_0 = _lse_sum(pr0, tail0_out, C1 - C0, 2048)
    s_1 = _lse_sum(pr1, tail1_out, V - C1, 2048)

    loss = _assemble(tgt.reshape(N, 1), s_h, s_0, s_1, hd2, flat, wh,
                     pr0, w0, pr1, w1)
    return loss.reshape(())
